# baseline (speedup 1.0000x reference)
import jax
import jax.numpy as jnp
from jax.experimental import pallas as pl

B, X, C_IN = 8, 16, 32
K, S, F = 3, 2, 64
OX = (X - K) // S + 1
NPOS = OX * OX * OX
CK = K * K * K * C_IN


def _im2col(x):
    slices = []
    for i in range(K):
        for j in range(K):
            for l in range(K):
                slices.append(x[:, i:i + S * (OX - 1) + 1:S,
                                  j:j + S * (OX - 1) + 1:S,
                                  l:l + S * (OX - 1) + 1:S, :])
    p = jnp.stack(slices, axis=0)
    p = p.transpose(2, 3, 4, 1, 0, 5)
    return p.reshape(NPOS, B, CK)


def _body(p_ref, ps_ref, loc_ref, rho_ref, eps_ref, b_ref, so_ref, o_ref):
    for j in range(OX):
        loc = loc_ref[0, 0, j].reshape(CK, F)
        w2 = (jax.nn.softplus(rho_ref[0, 0, j].reshape(CK, F))
              * eps_ref[0, 0, j].reshape(CK, F))
        m = jnp.dot(p_ref[j], loc, preferred_element_type=jnp.float32)
        pert = jnp.dot(ps_ref[j], w2, preferred_element_type=jnp.float32)
        o_ref[j] = m + b_ref[0, 0, j][None, :] + pert * so_ref[:, 0, 0, j, :]


def kernel(inputs, kernel_loc, kernel_rho, bias, eps, sign_in, sign_out):
    patches = _im2col(inputs)
    patches_s = _im2col(inputs * sign_in)

    grid = (OX * OX,)
    wspec = pl.BlockSpec((1, 1, OX, K, K, K, C_IN, F),
                         lambda i: (i // OX, i % OX, 0, 0, 0, 0, 0, 0))
    pspec = pl.BlockSpec((OX, B, CK), lambda i: (i, 0, 0))
    out = pl.pallas_call(
        _body,
        grid=grid,
        in_specs=[
            pspec, pspec, wspec, wspec, wspec,
            pl.BlockSpec((1, 1, OX, F), lambda i: (i // OX, i % OX, 0, 0)),
            pl.BlockSpec((B, 1, 1, OX, F), lambda i: (0, i // OX, i % OX, 0, 0)),
        ],
        out_specs=pl.BlockSpec((OX, B, F), lambda i: (i, 0, 0)),
        out_shape=jax.ShapeDtypeStruct((NPOS, B, F), jnp.float32),
    )(patches, patches_s, kernel_loc, kernel_rho, eps, bias, sign_out)

    return out.reshape(OX, OX, OX, B, F).transpose(3, 0, 1, 2, 4)


# speedup vs baseline: 49.7543x; 49.7543x over previous
import jax
import jax.numpy as jnp
from jax.experimental import pallas as pl

B, X, C_IN = 8, 16, 32
K, S, F = 3, 2, 64
OX = (X - K) // S + 1
NPOS = OX * OX * OX
CK = K * K * K * C_IN


def _gather_body(x_ref, s_ref, p_ref, ps_ref):
    i = pl.program_id(0)
    x = i // OX
    y = i % OX
    win = x_ref[:, pl.ds(2 * x, K), pl.ds(2 * y, K), :, :]
    sw = win * s_ref[:, pl.ds(2 * x, K), pl.ds(2 * y, K), :, :]
    for z in range(OX):
        p_ref[z] = win[:, :, :, 2 * z:2 * z + K, :].reshape(B, CK)
        ps_ref[z] = sw[:, :, :, 2 * z:2 * z + K, :].reshape(B, CK)


def _matmul_body(p_ref, ps_ref, loc_ref, rho_ref, eps_ref, b_ref, so_ref,
                 o_ref):
    for j in range(OX):
        loc = loc_ref[0, 0, j].reshape(CK, F)
        w2 = (jax.nn.softplus(rho_ref[0, 0, j].reshape(CK, F))
              * eps_ref[0, 0, j].reshape(CK, F))
        m = jnp.dot(p_ref[j], loc, preferred_element_type=jnp.float32)
        pert = jnp.dot(ps_ref[j], w2, preferred_element_type=jnp.float32)
        o_ref[j] = m + b_ref[0, 0, j][None, :] + pert * so_ref[:, 0, 0, j, :]


def kernel(inputs, kernel_loc, kernel_rho, bias, eps, sign_in, sign_out):
    full_in = pl.BlockSpec((B, X, X, X, C_IN), lambda i: (0, 0, 0, 0, 0))
    pspec = pl.BlockSpec((OX, B, CK), lambda i: (i, 0, 0))
    patches, patches_s = pl.pallas_call(
        _gather_body,
        grid=(OX * OX,),
        in_specs=[full_in, full_in],
        out_specs=[pspec, pspec],
        out_shape=[jax.ShapeDtypeStruct((NPOS, B, CK), jnp.float32)] * 2,
    )(inputs, sign_in)

    wspec = pl.BlockSpec((1, 1, OX, K, K, K, C_IN, F),
                         lambda i: (i // OX, i % OX, 0, 0, 0, 0, 0, 0))
    out = pl.pallas_call(
        _matmul_body,
        grid=(OX * OX,),
        in_specs=[
            pspec, pspec, wspec, wspec, wspec,
            pl.BlockSpec((1, 1, OX, F), lambda i: (i // OX, i % OX, 0, 0)),
            pl.BlockSpec((B, 1, 1, OX, F), lambda i: (0, i // OX, i % OX, 0, 0)),
        ],
        out_specs=pl.BlockSpec((OX, B, F), lambda i: (i, 0, 0)),
        out_shape=jax.ShapeDtypeStruct((NPOS, B, F), jnp.float32),
    )(patches, patches_s, kernel_loc, kernel_rho, eps, bias, sign_out)

    return out.reshape(OX, OX, OX, B, F).transpose(3, 0, 1, 2, 4)


# speedup vs baseline: 55.5477x; 1.1164x over previous
import jax
import jax.numpy as jnp
from jax.experimental import pallas as pl

B, X, C_IN = 8, 16, 32
K, S, F = 3, 2, 64
OX = (X - K) // S + 1
NPOS = OX * OX * OX
CK = K * K * K * C_IN


def _body(x_ref, s_ref, loc_ref, rho_ref, eps_ref, b_ref, so_ref, o_ref):
    i = pl.program_id(0)
    x = i // OX
    y = i % OX
    win = x_ref[:, pl.ds(2 * x, K), pl.ds(2 * y, K), :, :]
    sw = win * s_ref[:, pl.ds(2 * x, K), pl.ds(2 * y, K), :, :]
    for z in range(OX):
        p = win[:, :, :, 2 * z:2 * z + K, :].reshape(B, CK)
        ps = sw[:, :, :, 2 * z:2 * z + K, :].reshape(B, CK)
        loc = loc_ref[0, 0, z].reshape(CK, F)
        w2 = (jax.nn.softplus(rho_ref[0, 0, z].reshape(CK, F))
              * eps_ref[0, 0, z].reshape(CK, F))
        m = jnp.dot(p, loc, preferred_element_type=jnp.float32)
        pert = jnp.dot(ps, w2, preferred_element_type=jnp.float32)
        o_ref[z] = m + b_ref[0, 0, z][None, :] + pert * so_ref[:, 0, 0, z, :]


def kernel(inputs, kernel_loc, kernel_rho, bias, eps, sign_in, sign_out):
    full_in = pl.BlockSpec((B, X, X, X, C_IN), lambda i: (0, 0, 0, 0, 0))
    wspec = pl.BlockSpec((1, 1, OX, K, K, K, C_IN, F),
                         lambda i: (i // OX, i % OX, 0, 0, 0, 0, 0, 0))
    out = pl.pallas_call(
        _body,
        grid=(OX * OX,),
        in_specs=[
            full_in, full_in, wspec, wspec, wspec,
            pl.BlockSpec((1, 1, OX, F), lambda i: (i // OX, i % OX, 0, 0)),
            pl.BlockSpec((B, 1, 1, OX, F), lambda i: (0, i // OX, i % OX, 0, 0)),
        ],
        out_specs=pl.BlockSpec((OX, B, F), lambda i: (i, 0, 0)),
        out_shape=jax.ShapeDtypeStruct((NPOS, B, F), jnp.float32),
    )(inputs, sign_in, kernel_loc, kernel_rho, eps, bias, sign_out)

    return out.reshape(OX, OX, OX, B, F).transpose(3, 0, 1, 2, 4)
